# Initial kernel scaffold; baseline (speedup 1.0000x reference)
import jax
import jax.numpy as jnp
from jax.experimental import pallas as pl


def kernel(x, edge_index, edge_attr, W1, b1, g1, be1, a1, W2, b2, g2, be2, a2):
    raise NotImplementedError("write your pallas kernel here")


# speedup vs baseline: 2.5243x; 2.5243x over previous
import functools

import jax
import jax.numpy as jnp
from jax import lax
from jax.experimental import pallas as pl
from jax.experimental.pallas import tpu as pltpu
from jax.experimental.pallas import tpu_sc as plsc

_N = 10000
_E = 160000
_D = 256
_NPAD = 10240
_ROWS_T = 320
_K = 80
_CH = 8
_EROWS = 2048
_EPAD = _EROWS * _K
_CAP = 7680
_CROWS = _CAP // _K
_MESH = dict(core_axis_name="c", subcore_axis_name="s")
_SC_PARAMS = pltpu.CompilerParams(needs_layout_passes=False)


def _full16(v):
    return jnp.full((16,), v, jnp.int32)


@functools.partial(
    pl.kernel,
    out_type=[jax.ShapeDtypeStruct((32 * _CAP,), jnp.int32),
              jax.ShapeDtypeStruct((32 * _CAP,), jnp.float32),
              jax.ShapeDtypeStruct((32 * _CAP,), jnp.int32),
              jax.ShapeDtypeStruct((512,), jnp.int32),
              jax.ShapeDtypeStruct((_NPAD,), jnp.float32)],
    mesh=plsc.VectorSubcoreMesh(**_MESH),
    compiler_params=_SC_PARAMS,
    scratch_types=[
        pltpu.VMEM((_CH, _K), jnp.int32),
        pltpu.VMEM((_CH, _K), jnp.int32),
        pltpu.VMEM((_CH, _K), jnp.float32),
        pltpu.VMEM((_CAP + 16,), jnp.int32),
        pltpu.VMEM((_CAP + 16,), jnp.float32),
        pltpu.VMEM((_CAP + 16,), jnp.int32),
        pltpu.VMEM((16,), jnp.int32),
        pltpu.VMEM((_ROWS_T,), jnp.float32),
    ],
)
def _part_kernel(src2, dst2, w2, srcp, wp, lrp, cnt, deg,
                 sv, dv, wv, srcf, wf, lrf, cntv, degv):
    c = lax.axis_index("c")
    s = lax.axis_index("s")
    wid = s * 2 + c
    tb = wid * _ROWS_T
    lane = lax.iota(jnp.int32, 16)
    z16 = jnp.zeros((16,), jnp.float32)

    def zd(i, _):
        degv[pl.ds(i * 16, 16)] = z16
        return 0
    lax.fori_loop(0, _ROWS_T // 16, zd, 0)

    def chunk(ch, cur):
        pltpu.sync_copy(src2.at[pl.ds(ch * _CH, _CH)], sv)
        pltpu.sync_copy(dst2.at[pl.ds(ch * _CH, _CH)], dv)
        pltpu.sync_copy(w2.at[pl.ds(ch * _CH, _CH)], wv)
        for r in range(_CH):
            for j in range(_K // 16):
                sl = pl.ds(j * 16, 16)
                l16 = dv[r, sl] - tb
                m = (l16 >= 0) & (l16 < _ROWS_T)
                off = jnp.minimum(cur, _CAP - 16)
                plsc.store_compressed(srcf.at[pl.ds(off, 16)], sv[r, sl],
                                      mask=m)
                plsc.store_compressed(wf.at[pl.ds(off, 16)], wv[r, sl],
                                      mask=m)
                plsc.store_compressed(lrf.at[pl.ds(off, 16)], l16, mask=m)
                cur = cur + jnp.max(plsc.all_reduce_population_count(m))
        return cur
    cur = lax.fori_loop(0, _EROWS // _CH, chunk, jnp.int32(0))
    cur = jnp.minimum(cur, _CAP)

    k0 = cur // 16
    def fill(i, _):
        a0 = (k0 + i) * 16
        ge = a0 + lane >= cur
        srcf[pl.ds(a0, 16)] = jnp.where(
            ge, ((a0 + lane) * 37) & 8191, srcf[pl.ds(a0, 16)])
        wf[pl.ds(a0, 16)] = jnp.where(ge, 0.0, wf[pl.ds(a0, 16)])
        lrf[pl.ds(a0, 16)] = jnp.where(ge, 0, lrf[pl.ds(a0, 16)])
        return 0
    lax.fori_loop(0, _CAP // 16 - k0, fill, 0)

    def dacc(e, _):
        ehi = (e // 16) * 16
        em = lane == (e - ehi)
        lr = jnp.max(jnp.where(em, lrf[pl.ds(ehi, 16)], 0))
        wsc = jnp.max(jnp.where(em, wf[pl.ds(ehi, 16)], -3.0e38))
        drow = (lr // 16) * 16
        degv[pl.ds(drow, 16)] = degv[pl.ds(drow, 16)] + \
            jnp.where(lane == (lr - drow), wsc, 0.0)
        return 0
    lax.fori_loop(0, _CAP, dacc, 0)

    cntv[...] = jnp.broadcast_to(cur, (16,))
    pltpu.sync_copy(srcf.at[pl.ds(0, _CAP)], srcp.at[pl.ds(wid * _CAP, _CAP)])
    pltpu.sync_copy(wf.at[pl.ds(0, _CAP)], wp.at[pl.ds(wid * _CAP, _CAP)])
    pltpu.sync_copy(lrf.at[pl.ds(0, _CAP)], lrp.at[pl.ds(wid * _CAP, _CAP)])
    pltpu.sync_copy(cntv, cnt.at[pl.ds(wid * 16, 16)])
    pltpu.sync_copy(degv, deg.at[pl.ds(tb, _ROWS_T)])


@functools.partial(
    pl.kernel,
    out_type=jax.ShapeDtypeStruct((32 * _CROWS, _K), jnp.float32),
    mesh=plsc.VectorSubcoreMesh(**_MESH),
    compiler_params=_SC_PARAMS,
    scratch_types=[
        pltpu.VMEM((_NPAD,), jnp.float32),
        pltpu.VMEM((_CH, _K), jnp.int32),
        pltpu.VMEM((_CH, _K), jnp.float32),
        pltpu.VMEM((_CH, _K), jnp.int32),
        pltpu.VMEM((_CH, _K), jnp.float32),
    ],
)
def _coef_kernel(srcp, wp, lrp, dis, out, disv, sv, wv, lv, cvv):
    c = lax.axis_index("c")
    s = lax.axis_index("s")
    wid = s * 2 + c
    tb = wid * _ROWS_T
    pltpu.sync_copy(dis, disv)

    def group(g, _):
        r0 = wid * _CROWS + g * _CH
        pltpu.sync_copy(srcp.at[pl.ds(r0, _CH)], sv)
        pltpu.sync_copy(wp.at[pl.ds(r0, _CH)], wv)
        pltpu.sync_copy(lrp.at[pl.ds(r0, _CH)], lv)
        for r in range(_CH):
            for j in range(_K // 16):
                sl = pl.ds(j * 16, 16)
                cvv[r, sl] = plsc.load_gather(disv, [sv[r, sl]]) * wv[r, sl] \
                    * plsc.load_gather(disv, [lv[r, sl] + tb])
        pltpu.sync_copy(cvv, out.at[pl.ds(r0, _CH)])
        return 0
    lax.fori_loop(0, _CROWS // _CH, group, 0)


@functools.partial(
    pl.kernel,
    out_type=jax.ShapeDtypeStruct((_NPAD, _D), jnp.float32),
    mesh=plsc.VectorSubcoreMesh(**_MESH),
    compiler_params=_SC_PARAMS,
    scratch_types=[
        pltpu.VMEM((_CH, _K), jnp.int32),
        pltpu.VMEM((_CH, _K), jnp.float32),
        pltpu.VMEM((_CH, _K), jnp.int32),
        pltpu.VMEM((_ROWS_T,), jnp.float32),
        pltpu.VMEM((16,), jnp.int32),
        pltpu.VMEM((_K, _D), jnp.float32),
        pltpu.VMEM((_ROWS_T, _D), jnp.float32),
        pltpu.SemaphoreType.DMA,
    ],
)
def _msg_kernel(xl, srcp, cp, lrp, cnt, isd, out,
                sv, cvv, lv, isdv, cntv, rows, acc, sem):
    c = lax.axis_index("c")
    s = lax.axis_index("s")
    wid = s * 2 + c
    tb = wid * _ROWS_T
    lane = lax.iota(jnp.int32, 16)

    pltpu.sync_copy(isd.at[pl.ds(tb, _ROWS_T)], isdv)
    pltpu.sync_copy(cnt.at[pl.ds(wid * 16, 16)], cntv)
    nb = jnp.max((cntv[...] + (_K - 1)) // _K)
    ng = (nb + (_CH - 1)) // _CH

    def init_chunk(k, _):
        pltpu.async_copy(xl.at[pl.ds(tb + k * _K, _K)], rows, sem).wait()

        def srow(j, _):
            iv = plsc.load_gather(isdv, [_full16(k * _K + j)])
            for t in range(_D // 16):
                sl = pl.ds(t * 16, 16)
                acc[k * _K + j, sl] = rows[j, sl] * iv
            return 0
        lax.fori_loop(0, _K, srow, 0)
        return 0
    lax.fori_loop(0, _ROWS_T // _K, init_chunk, 0)

    def group(g, _):
        r0 = wid * _CROWS + g * _CH
        pltpu.sync_copy(srcp.at[pl.ds(r0, _CH)], sv)
        pltpu.sync_copy(cp.at[pl.ds(r0, _CH)], cvv)
        pltpu.sync_copy(lrp.at[pl.ds(r0, _CH)], lv)
        for r in range(_CH):
            @pl.when(g * _CH + r < nb)
            def _():
                pltpu.async_copy(xl.at[sv.at[r]], rows, sem).wait()

                def erow(j, _):
                    jhi = (j // 16) * 16
                    jm = lane == (j - jhi)
                    lr = jnp.max(jnp.where(jm, lv[r, pl.ds(jhi, 16)], 0))
                    cj = plsc.load_gather(cvv, [_full16(r), _full16(j)])
                    for t in range(_D // 16):
                        sl = pl.ds(t * 16, 16)
                        acc[lr, sl] = acc[lr, sl] + rows[j, sl] * cj
                    return 0
                lax.fori_loop(0, _K, erow, 0)
        return 0
    lax.fori_loop(0, ng, group, 0)

    pltpu.sync_copy(acc, out.at[pl.ds(tb, _ROWS_T)])


def _prep_body(deg_ref, dis_ref, isd_ref):
    d = deg_ref[...] + 1.0
    dis_ref[...] = lax.rsqrt(d)
    isd_ref[...] = 1.0 / d


def _prep(deg2):
    return pl.pallas_call(
        _prep_body,
        out_shape=[jax.ShapeDtypeStruct((1, _NPAD), jnp.float32),
                   jax.ShapeDtypeStruct((1, _NPAD), jnp.float32)],
    )(deg2)


def _mm_body(x_ref, w_ref, o_ref):
    o_ref[...] = lax.dot_general(x_ref[...], w_ref[...],
                                 (((1,), (1,)), ((), ())),
                                 preferred_element_type=jnp.float32)


def _mm(x, w):
    return pl.pallas_call(
        _mm_body,
        grid=(5,),
        in_specs=[pl.BlockSpec((2048, _D), lambda i: (i, 0)),
                  pl.BlockSpec((_D, _D), lambda i: (0, 0))],
        out_specs=pl.BlockSpec((2048, _D), lambda i: (i, 0)),
        out_shape=jax.ShapeDtypeStruct((_NPAD, _D), jnp.float32),
    )(x, w)


def _stats_body(acc_ref, g_ref, be_ref, st_ref, ssum, ssq):
    i = pl.program_id(0)

    @pl.when(i == 0)
    def _():
        ssum[...] = jnp.zeros_like(ssum)
        ssq[...] = jnp.zeros_like(ssq)

    blk = acc_ref[...]
    ssum[...] += jnp.sum(blk, 0, keepdims=True)
    ssq[...] += jnp.sum(blk * blk, 0, keepdims=True)

    @pl.when(i == pl.num_programs(0) - 1)
    def _():
        m = ssum[...] * (1.0 / _N)
        v = ssq[...] * (1.0 / _N) - m * m
        sc = g_ref[...] * lax.rsqrt(v + 1e-5)
        st_ref[0:1, :] = sc
        st_ref[1:2, :] = be_ref[...] - m * sc


def _stats(acc, g, be):
    return pl.pallas_call(
        _stats_body,
        grid=(5,),
        in_specs=[pl.BlockSpec((2000, _D), lambda i: (i, 0)),
                  pl.BlockSpec((1, _D), lambda i: (0, 0)),
                  pl.BlockSpec((1, _D), lambda i: (0, 0))],
        out_specs=pl.BlockSpec((2, _D), lambda i: (0, 0)),
        out_shape=jax.ShapeDtypeStruct((2, _D), jnp.float32),
        scratch_shapes=[pltpu.VMEM((1, _D), jnp.float32),
                        pltpu.VMEM((1, _D), jnp.float32)],
    )(acc, g, be)


def _apply_mm_body(acc_ref, st_ref, a_ref, w_ref, o_ref):
    h = acc_ref[...] * st_ref[0:1, :] + st_ref[1:2, :]
    h = jnp.where(h >= 0, h, a_ref[0, 0] * h)
    o_ref[...] = lax.dot_general(h, w_ref[...], (((1,), (1,)), ((), ())),
                                 preferred_element_type=jnp.float32)


def _apply_mm(acc, st, a, w):
    return pl.pallas_call(
        _apply_mm_body,
        grid=(5,),
        in_specs=[pl.BlockSpec((2048, _D), lambda i: (i, 0)),
                  pl.BlockSpec((2, _D), lambda i: (0, 0)),
                  pl.BlockSpec(memory_space=pltpu.SMEM),
                  pl.BlockSpec((_D, _D), lambda i: (0, 0))],
        out_specs=pl.BlockSpec((2048, _D), lambda i: (i, 0)),
        out_shape=jax.ShapeDtypeStruct((_NPAD, _D), jnp.float32),
    )(acc, st, a, w)


def _apply_body(acc_ref, st_ref, a_ref, o_ref):
    h = acc_ref[...] * st_ref[0:1, :] + st_ref[1:2, :]
    o_ref[...] = jnp.where(h >= 0, h, a_ref[0, 0] * h)


def _apply(acc, st, a):
    return pl.pallas_call(
        _apply_body,
        grid=(5,),
        in_specs=[pl.BlockSpec((2000, _D), lambda i: (i, 0)),
                  pl.BlockSpec((2, _D), lambda i: (0, 0)),
                  pl.BlockSpec(memory_space=pltpu.SMEM)],
        out_specs=pl.BlockSpec((2000, _D), lambda i: (i, 0)),
        out_shape=jax.ShapeDtypeStruct((_N, _D), jnp.float32),
    )(acc, st, a)


def kernel(x, edge_index, edge_attr, W1, b1, g1, be1, a1, W2, b2, g2, be2, a2):
    npad = _EPAD - _E
    src2 = jnp.concatenate(
        [edge_index[0], jnp.zeros((npad,), jnp.int32)]).reshape(_EROWS, _K)
    dst2 = jnp.concatenate(
        [edge_index[1], jnp.full((npad,), 1 << 28, jnp.int32)]
    ).reshape(_EROWS, _K)
    w2e = jnp.concatenate(
        [edge_attr, jnp.zeros((npad,), jnp.float32)]).reshape(_EROWS, _K)

    srcp, wp, lrp, cnt, deg = _part_kernel(src2, dst2, w2e)
    dis, isd = _prep(deg.reshape(1, _NPAD))
    dis = dis.reshape(_NPAD)
    isd = isd.reshape(_NPAD)
    srcp = srcp.reshape(32 * _CROWS, _K)
    wp = wp.reshape(32 * _CROWS, _K)
    lrp = lrp.reshape(32 * _CROWS, _K)
    cp = _coef_kernel(srcp, wp, lrp, dis)

    xl1 = _mm(x, W1)
    acc1 = _msg_kernel(xl1, srcp, cp, lrp, cnt, isd)
    st1 = _stats(acc1, g1.reshape(1, _D), be1.reshape(1, _D))

    xl2 = _apply_mm(acc1, st1, a1.reshape(1, 1), W2)
    acc2 = _msg_kernel(xl2, srcp, cp, lrp, cnt, isd)
    st2 = _stats(acc2, g2.reshape(1, _D), be2.reshape(1, _D))

    return _apply(acc2, st2, a2.reshape(1, 1))


# speedup vs baseline: 3.7955x; 1.5036x over previous
import functools

import jax
import jax.numpy as jnp
from jax import lax
from jax.experimental import pallas as pl
from jax.experimental.pallas import tpu as pltpu
from jax.experimental.pallas import tpu_sc as plsc

_N = 10000
_E = 160000
_D = 256
_NPAD = 10240
_ROWS_T = 320
_K = 80
_CH = 8
_EROWS = 2048
_EPAD = _EROWS * _K
_CAP = 6400
_CROWS = _CAP // _K
_MESH = dict(core_axis_name="c", subcore_axis_name="s")
_SC_PARAMS = pltpu.CompilerParams(needs_layout_passes=False)


def _full16(v):
    return jnp.full((16,), v, jnp.int32)


@functools.partial(
    pl.kernel,
    out_type=[jax.ShapeDtypeStruct((32 * _CAP,), jnp.int32),
              jax.ShapeDtypeStruct((32 * _CAP,), jnp.float32),
              jax.ShapeDtypeStruct((32 * _CAP,), jnp.int32),
              jax.ShapeDtypeStruct((512,), jnp.int32),
              jax.ShapeDtypeStruct((_NPAD,), jnp.float32)],
    mesh=plsc.VectorSubcoreMesh(**_MESH),
    compiler_params=_SC_PARAMS,
    scratch_types=[
        pltpu.VMEM((64, _K), jnp.int32),
        pltpu.VMEM((64, _K), jnp.int32),
        pltpu.VMEM((64, _K), jnp.float32),
        pltpu.VMEM((_CAP + 16,), jnp.int32),
        pltpu.VMEM((_CAP + 16,), jnp.float32),
        pltpu.VMEM((_CAP + 16,), jnp.int32),
        pltpu.VMEM((16,), jnp.int32),
        pltpu.VMEM((_ROWS_T,), jnp.float32),
    ],
)
def _part_kernel(src2, dst2, w2, srcp, wp, lrp, cnt, deg,
                 sv, dv, wv, srcf, wf, lrf, cntv, degv):
    c = lax.axis_index("c")
    s = lax.axis_index("s")
    wid = s * 2 + c
    tb = wid * _ROWS_T
    lane = lax.iota(jnp.int32, 16)
    z16 = jnp.zeros((16,), jnp.float32)

    def zd(i, _):
        degv[pl.ds(i * 16, 16)] = z16
        return 0
    lax.fori_loop(0, _ROWS_T // 16, zd, 0)

    def chunk(ch, cur):
        pltpu.sync_copy(src2.at[pl.ds(ch * 64, 64)], sv)
        pltpu.sync_copy(dst2.at[pl.ds(ch * 64, 64)], dv)
        pltpu.sync_copy(w2.at[pl.ds(ch * 64, 64)], wv)

        def row(r, cur):
            for j in range(_K // 16):
                sl = pl.ds(j * 16, 16)
                l16 = dv[r, sl] - tb
                m = (l16 >= 0) & (l16 < _ROWS_T)
                off = jnp.minimum(cur, _CAP - 16)
                plsc.store_compressed(srcf.at[pl.ds(off, 16)], sv[r, sl],
                                      mask=m)
                plsc.store_compressed(wf.at[pl.ds(off, 16)], wv[r, sl],
                                      mask=m)
                plsc.store_compressed(lrf.at[pl.ds(off, 16)], l16, mask=m)
                cur = cur + jnp.max(plsc.all_reduce_population_count(m))
            return cur
        return lax.fori_loop(0, 64, row, cur)
    cur = lax.fori_loop(0, _EROWS // 64, chunk, jnp.int32(0))
    cur = jnp.minimum(cur, _CAP)

    k0 = cur // 16
    def fill(i, _):
        a0 = (k0 + i) * 16
        ge = a0 + lane >= cur
        srcf[pl.ds(a0, 16)] = jnp.where(
            ge, ((a0 + lane) * 37) & 8191, srcf[pl.ds(a0, 16)])
        wf[pl.ds(a0, 16)] = jnp.where(ge, 0.0, wf[pl.ds(a0, 16)])
        lrf[pl.ds(a0, 16)] = jnp.where(ge, 0, lrf[pl.ds(a0, 16)])
        return 0
    lax.fori_loop(0, _CAP // 16 - k0, fill, 0)

    def dacc(e, _):
        ehi = (e // 16) * 16
        em = lane == (e - ehi)
        lr = jnp.max(jnp.where(em, lrf[pl.ds(ehi, 16)], 0))
        wsc = jnp.max(jnp.where(em, wf[pl.ds(ehi, 16)], -3.0e38))
        drow = (lr // 16) * 16
        plsc.addupdate(degv.at[pl.ds(drow, 16)],
                       jnp.where(lane == (lr - drow), wsc, 0.0))
        return 0
    lax.fori_loop(0, _CAP, dacc, 0)

    cntv[...] = jnp.broadcast_to(cur, (16,))
    pltpu.sync_copy(srcf.at[pl.ds(0, _CAP)], srcp.at[pl.ds(wid * _CAP, _CAP)])
    pltpu.sync_copy(wf.at[pl.ds(0, _CAP)], wp.at[pl.ds(wid * _CAP, _CAP)])
    pltpu.sync_copy(lrf.at[pl.ds(0, _CAP)], lrp.at[pl.ds(wid * _CAP, _CAP)])
    pltpu.sync_copy(cntv, cnt.at[pl.ds(wid * 16, 16)])
    pltpu.sync_copy(degv, deg.at[pl.ds(tb, _ROWS_T)])


@functools.partial(
    pl.kernel,
    out_type=jax.ShapeDtypeStruct((32 * _CROWS, _K), jnp.float32),
    mesh=plsc.VectorSubcoreMesh(**_MESH),
    compiler_params=_SC_PARAMS,
    scratch_types=[
        pltpu.VMEM((_NPAD,), jnp.float32),
        pltpu.VMEM((_CH, _K), jnp.int32),
        pltpu.VMEM((_CH, _K), jnp.float32),
        pltpu.VMEM((_CH, _K), jnp.int32),
        pltpu.VMEM((_CH, _K), jnp.float32),
    ],
)
def _coef_kernel(srcp, wp, lrp, dis, out, disv, sv, wv, lv, cvv):
    c = lax.axis_index("c")
    s = lax.axis_index("s")
    wid = s * 2 + c
    tb = wid * _ROWS_T
    pltpu.sync_copy(dis, disv)

    def group(g, _):
        r0 = wid * _CROWS + g * _CH
        pltpu.sync_copy(srcp.at[pl.ds(r0, _CH)], sv)
        pltpu.sync_copy(wp.at[pl.ds(r0, _CH)], wv)
        pltpu.sync_copy(lrp.at[pl.ds(r0, _CH)], lv)
        for r in range(_CH):
            for j in range(_K // 16):
                sl = pl.ds(j * 16, 16)
                cvv[r, sl] = plsc.load_gather(disv, [sv[r, sl]]) * wv[r, sl] \
                    * plsc.load_gather(disv, [lv[r, sl] + tb])
        pltpu.sync_copy(cvv, out.at[pl.ds(r0, _CH)])
        return 0
    lax.fori_loop(0, _CROWS // _CH, group, 0)


@functools.partial(
    pl.kernel,
    out_type=jax.ShapeDtypeStruct((_NPAD, _D), jnp.float32),
    mesh=plsc.VectorSubcoreMesh(**_MESH),
    compiler_params=_SC_PARAMS,
    scratch_types=[
        pltpu.VMEM((_CH, _K), jnp.int32),
        pltpu.VMEM((_CH, _K), jnp.float32),
        pltpu.VMEM((_CH, _K), jnp.int32),
        pltpu.VMEM((_K,), jnp.float32),
        pltpu.VMEM((16,), jnp.int32),
        pltpu.VMEM((_K, _D), jnp.float32),
        pltpu.VMEM((_K, _D), jnp.float32),
        pltpu.VMEM((_ROWS_T, _D), jnp.float32),
        pltpu.SemaphoreType.DMA,
        pltpu.SemaphoreType.DMA,
    ],
)
def _msg_kernel(xl, srcp, cp, lrp, cnt, isd, out,
                sv, cvv, lv, isdv, cntv, rowsA, rowsB, acc, semA, semB):
    c = lax.axis_index("c")
    s = lax.axis_index("s")
    wid = s * 2 + c
    tb = wid * _ROWS_T
    lane = lax.iota(jnp.int32, 16)
    bufs = (rowsA, rowsB)
    sems = (semA, semB)

    pltpu.sync_copy(cnt.at[pl.ds(wid * 16, 16)], cntv)
    nb = jnp.max((cntv[...] + (_K - 1)) // _K)
    ng = (nb + (_CH - 1)) // _CH

    def init_chunk(k, _):
        pltpu.async_copy(xl.at[pl.ds(tb + k * _K, _K)], rowsA, semA).wait()
        pltpu.sync_copy(isd.at[pl.ds(tb + k * _K, _K)], isdv)

        def srow(j, _):
            iv = plsc.load_gather(isdv, [_full16(j)])
            for t in range(_D // 16):
                sl = pl.ds(t * 16, 16)
                acc[k * _K + j, sl] = rowsA[j, sl] * iv
            return 0
        lax.fori_loop(0, _K, srow, 0)
        return 0
    lax.fori_loop(0, _ROWS_T // _K, init_chunk, 0)

    def group(g, _):
        r0 = wid * _CROWS + g * _CH
        pltpu.sync_copy(srcp.at[pl.ds(r0, _CH)], sv)
        pltpu.sync_copy(cp.at[pl.ds(r0, _CH)], cvv)
        pltpu.sync_copy(lrp.at[pl.ds(r0, _CH)], lv)

        @pl.when(g * _CH < nb)
        def _():
            pltpu.async_copy(xl.at[sv.at[0]], bufs[0], sems[0])

        for r in range(_CH):
            b = g * _CH + r
            p = r % 2

            @pl.when(b < nb)
            def _(r=r, p=p, b=b):
                pltpu.make_async_copy(
                    xl.at[pl.ds(0, _K)], bufs[p], sems[p]).wait()
                rowsP = bufs[p]

                if r < _CH - 1:
                    @pl.when(b + 1 < nb)
                    def _():
                        pltpu.async_copy(
                            xl.at[sv.at[r + 1]], bufs[1 - p], sems[1 - p])

                def erow(j, _):
                    jhi = (j // 16) * 16
                    jm = lane == (j - jhi)
                    lr = jnp.max(jnp.where(jm, lv[r, pl.ds(jhi, 16)], 0))
                    cj = plsc.load_gather(cvv, [_full16(r), _full16(j)])
                    for t in range(_D // 16):
                        sl = pl.ds(t * 16, 16)
                        plsc.addupdate(acc.at[lr, sl], rowsP[j, sl] * cj)
                    return 0
                lax.fori_loop(0, _K, erow, 0)
        return 0
    lax.fori_loop(0, ng, group, 0)

    pltpu.sync_copy(acc, out.at[pl.ds(tb, _ROWS_T)])


def _prep_body(deg_ref, dis_ref, isd_ref):
    d = deg_ref[...] + 1.0
    dis_ref[...] = lax.rsqrt(d)
    isd_ref[...] = 1.0 / d


def _prep(deg2):
    return pl.pallas_call(
        _prep_body,
        out_shape=[jax.ShapeDtypeStruct((1, _NPAD), jnp.float32),
                   jax.ShapeDtypeStruct((1, _NPAD), jnp.float32)],
    )(deg2)


def _mm_body(x_ref, w_ref, o_ref):
    o_ref[...] = lax.dot_general(x_ref[...], w_ref[...],
                                 (((1,), (1,)), ((), ())),
                                 preferred_element_type=jnp.float32)


def _mm(x, w):
    return pl.pallas_call(
        _mm_body,
        grid=(5,),
        in_specs=[pl.BlockSpec((2048, _D), lambda i: (i, 0)),
                  pl.BlockSpec((_D, _D), lambda i: (0, 0))],
        out_specs=pl.BlockSpec((2048, _D), lambda i: (i, 0)),
        out_shape=jax.ShapeDtypeStruct((_NPAD, _D), jnp.float32),
    )(x, w)


def _stats_body(acc_ref, g_ref, be_ref, st_ref, ssum, ssq):
    i = pl.program_id(0)

    @pl.when(i == 0)
    def _():
        ssum[...] = jnp.zeros_like(ssum)
        ssq[...] = jnp.zeros_like(ssq)

    blk = acc_ref[...]
    ssum[...] += jnp.sum(blk, 0, keepdims=True)
    ssq[...] += jnp.sum(blk * blk, 0, keepdims=True)

    @pl.when(i == pl.num_programs(0) - 1)
    def _():
        m = ssum[...] * (1.0 / _N)
        v = ssq[...] * (1.0 / _N) - m * m
        sc = g_ref[...] * lax.rsqrt(v + 1e-5)
        st_ref[0:1, :] = sc
        st_ref[1:2, :] = be_ref[...] - m * sc


def _stats(acc, g, be):
    return pl.pallas_call(
        _stats_body,
        grid=(5,),
        in_specs=[pl.BlockSpec((2000, _D), lambda i: (i, 0)),
                  pl.BlockSpec((1, _D), lambda i: (0, 0)),
                  pl.BlockSpec((1, _D), lambda i: (0, 0))],
        out_specs=pl.BlockSpec((2, _D), lambda i: (0, 0)),
        out_shape=jax.ShapeDtypeStruct((2, _D), jnp.float32),
        scratch_shapes=[pltpu.VMEM((1, _D), jnp.float32),
                        pltpu.VMEM((1, _D), jnp.float32)],
    )(acc, g, be)


def _apply_mm_body(acc_ref, st_ref, a_ref, w_ref, o_ref):
    h = acc_ref[...] * st_ref[0:1, :] + st_ref[1:2, :]
    h = jnp.where(h >= 0, h, a_ref[0, 0] * h)
    o_ref[...] = lax.dot_general(h, w_ref[...], (((1,), (1,)), ((), ())),
                                 preferred_element_type=jnp.float32)


def _apply_mm(acc, st, a, w):
    return pl.pallas_call(
        _apply_mm_body,
        grid=(5,),
        in_specs=[pl.BlockSpec((2048, _D), lambda i: (i, 0)),
                  pl.BlockSpec((2, _D), lambda i: (0, 0)),
                  pl.BlockSpec(memory_space=pltpu.SMEM),
                  pl.BlockSpec((_D, _D), lambda i: (0, 0))],
        out_specs=pl.BlockSpec((2048, _D), lambda i: (i, 0)),
        out_shape=jax.ShapeDtypeStruct((_NPAD, _D), jnp.float32),
    )(acc, st, a, w)


def _apply_body(acc_ref, st_ref, a_ref, o_ref):
    h = acc_ref[...] * st_ref[0:1, :] + st_ref[1:2, :]
    o_ref[...] = jnp.where(h >= 0, h, a_ref[0, 0] * h)


def _apply(acc, st, a):
    return pl.pallas_call(
        _apply_body,
        grid=(5,),
        in_specs=[pl.BlockSpec((2000, _D), lambda i: (i, 0)),
                  pl.BlockSpec((2, _D), lambda i: (0, 0)),
                  pl.BlockSpec(memory_space=pltpu.SMEM)],
        out_specs=pl.BlockSpec((2000, _D), lambda i: (i, 0)),
        out_shape=jax.ShapeDtypeStruct((_N, _D), jnp.float32),
    )(acc, st, a)


def kernel(x, edge_index, edge_attr, W1, b1, g1, be1, a1, W2, b2, g2, be2, a2):
    npad = _EPAD - _E
    src2 = jnp.concatenate(
        [edge_index[0], jnp.zeros((npad,), jnp.int32)]).reshape(_EROWS, _K)
    dst2 = jnp.concatenate(
        [edge_index[1], jnp.full((npad,), 1 << 28, jnp.int32)]
    ).reshape(_EROWS, _K)
    w2e = jnp.concatenate(
        [edge_attr, jnp.zeros((npad,), jnp.float32)]).reshape(_EROWS, _K)

    srcp, wp, lrp, cnt, deg = _part_kernel(src2, dst2, w2e)
    dis, isd = _prep(deg.reshape(1, _NPAD))
    dis = dis.reshape(_NPAD)
    isd = isd.reshape(_NPAD)
    srcp = srcp.reshape(32 * _CROWS, _K)
    wp = wp.reshape(32 * _CROWS, _K)
    lrp = lrp.reshape(32 * _CROWS, _K)
    cp = _coef_kernel(srcp, wp, lrp, dis)

    xl1 = _mm(x, W1)
    acc1 = _msg_kernel(xl1, srcp, cp, lrp, cnt, isd)
    st1 = _stats(acc1, g1.reshape(1, _D), be1.reshape(1, _D))

    xl2 = _apply_mm(acc1, st1, a1.reshape(1, 1), W2)
    acc2 = _msg_kernel(xl2, srcp, cp, lrp, cnt, isd)
    st2 = _stats(acc2, g2.reshape(1, _D), be2.reshape(1, _D))

    return _apply(acc2, st2, a2.reshape(1, 1))
